# baseline (speedup 1.0000x reference)
import jax
import jax.numpy as jnp
from jax import lax
from jax.experimental import pallas as pl
from jax.experimental.pallas import tpu as pltpu

B, C, H, W, N = 16, 80, 100, 100, 32
LAMBDA_L1, LAMBDA_GIOU, POS_IOU_THR = 1.0, 2.0, 0.3
R_LO, R_HI = 2, 1
G = 32


def _reduce2(x, op):
    r = op(x, axis=0, keepdims=True)
    return op(r, axis=1, keepdims=True)


def _one_gt(pred_ref, gx0, gy0, gx1, gy1):
    x0 = pred_ref[0, 0, 0::4, :]
    y0 = pred_ref[0, 0, 1::4, :]
    x1 = pred_ref[0, 0, 2::4, :]
    y1 = pred_ref[0, 0, 3::4, :]

    area_a = (x1 - x0) * (y1 - y0)
    area_b = (gx1 - gx0) * (gy1 - gy0)
    ltx = jnp.maximum(x0, gx0)
    lty = jnp.maximum(y0, gy0)
    rbx = jnp.minimum(x1, gx1)
    rby = jnp.minimum(y1, gy1)
    iw = jnp.maximum(rbx - ltx, 0.0)
    ih = jnp.maximum(rby - lty, 0.0)
    inter = iw * ih
    union = area_a + area_b - inter
    iou = inter / union
    cx = jnp.minimum(x0, gx0)
    cy = jnp.minimum(y0, gy0)
    dx = jnp.maximum(x1, gx1)
    dy = jnp.maximum(y1, gy1)
    area_c = (dx - cx) * (dy - cy)
    g = iou - (area_c - union) / area_c
    l1 = ((ltx - cx) + (lty - cy) + (dx - rbx) + (dy - rby)) * 0.25

    m = _reduce2(g, jnp.max)
    rows_f = lax.broadcasted_iota(jnp.int32, (H, W), 0).astype(jnp.float32)
    cols_f = lax.broadcasted_iota(jnp.int32, (H, W), 1).astype(jnp.float32)
    key = rows_f * 128.0 + cols_f
    cand = jnp.where(g == m, key, 3.4e38)
    kmin = _reduce2(cand, jnp.min)
    mi = jnp.floor(kmin * (1.0 / 128.0))
    mj = kmin - mi * 128.0

    rmask = (rows_f >= mi - float(R_LO)) & (rows_f <= jnp.minimum(mi + float(R_HI), float(H - 1)))
    cmask = (cols_f >= mj - float(R_LO)) & (cols_f <= jnp.minimum(mj + float(R_HI), float(W - 1)))
    sel = rmask & cmask

    s_l1 = _reduce2(jnp.where(sel, l1, 0.0), jnp.sum)
    rn = jnp.minimum(mi + float(R_HI), float(H - 1)) - jnp.maximum(mi - float(R_LO), 0.0) + 1.0
    cn = jnp.minimum(mj + float(R_HI), float(W - 1)) - jnp.maximum(mj - float(R_LO), 0.0) + 1.0
    s_c = rn * cn
    s_g = s_c - _reduce2(jnp.where(sel, g, 0.0), jnp.sum)

    valid = jnp.where(m > POS_IOU_THR, 1.0, 0.0)

    lane = lax.broadcasted_iota(jnp.int32, (1, 128), 1)
    return (jnp.where(lane == 0, 1.0, 0.0) * s_l1
            + jnp.where(lane == 1, 1.0, 0.0) * s_g
            + jnp.where(lane == 2, 1.0, 0.0) * s_c) * valid


def _loss_kernel(labels_ref, *refs):
    preds, gt_ref, out_ref = refs[:G], refs[G], refs[G + 1]
    s = pl.program_id(0)

    rows = []
    for i, p in enumerate(preds):
        base = (s * G + i) * 4
        rows.append(_one_gt(p, gt_ref[base + 0], gt_ref[base + 1],
                            gt_ref[base + 2], gt_ref[base + 3]))
    out_ref[...] = jnp.concatenate(rows, axis=0).reshape(1, G, 128)


def kernel(pred_boxes, gt_boxes, gt_labels):
    pred_t = jnp.moveaxis(pred_boxes, -1, 3).reshape(B, C, 4 * H, W)
    gt_flat = gt_boxes.reshape(-1)
    labels = gt_labels.astype(jnp.int32)

    labels_flat = labels.reshape(-1)

    pred_spec = [
        pl.BlockSpec((1, 1, 4 * H, W),
                     (lambda s, labels, i=i:
                      ((s * G + i) // N, labels[s * G + i], 0, 0)))
        for i in range(G)
    ]

    out = pl.pallas_call(
        _loss_kernel,
        grid_spec=pltpu.PrefetchScalarGridSpec(
            num_scalar_prefetch=1,
            grid=(B * N // G,),
            in_specs=pred_spec + [pl.BlockSpec(memory_space=pltpu.SMEM)],
            out_specs=pl.BlockSpec((1, G, 128),
                                   lambda s, labels: (s, 0, 0)),
        ),
        out_shape=jax.ShapeDtypeStruct((B * N // G, G, 128), jnp.float32),
        compiler_params=pltpu.CompilerParams(
            dimension_semantics=("arbitrary",),
        ),
        name="gtclass_loc_loss",
    )(labels_flat, *([pred_t] * G), gt_flat)

    l1_sum = jnp.sum(out[:, :, 0])
    g_sum = jnp.sum(out[:, :, 1])
    n_pos = jnp.sum(out[:, :, 2])
    denom = jnp.maximum(n_pos, 1.0)
    return LAMBDA_L1 * (l1_sum / denom) + LAMBDA_GIOU * (g_sum / denom)


# speedup vs baseline: 1.0197x; 1.0197x over previous
import jax
import jax.numpy as jnp
from jax import lax
from jax.experimental import pallas as pl
from jax.experimental.pallas import tpu as pltpu

B, C, H, W, N = 16, 80, 100, 100, 32
LAMBDA_L1, LAMBDA_GIOU, POS_IOU_THR = 1.0, 2.0, 0.3
R_LO, R_HI = 2, 1
G = 32


def _reduce2(x, op):
    r = op(x, axis=0, keepdims=True)
    return op(r, axis=1, keepdims=True)


def _one_gt(pred_ref, gx0, gy0, gx1, gy1):
    x0 = pred_ref[0, 0, 0::4, :]
    y0 = pred_ref[0, 0, 1::4, :]
    x1 = pred_ref[0, 0, 2::4, :]
    y1 = pred_ref[0, 0, 3::4, :]

    area_a = (x1 - x0) * (y1 - y0)
    area_b = (gx1 - gx0) * (gy1 - gy0)
    ltx = jnp.maximum(x0, gx0)
    lty = jnp.maximum(y0, gy0)
    rbx = jnp.minimum(x1, gx1)
    rby = jnp.minimum(y1, gy1)
    iw = jnp.maximum(rbx - ltx, 0.0)
    ih = jnp.maximum(rby - lty, 0.0)
    inter = iw * ih
    union = area_a + area_b - inter
    iou = inter / union
    cx = jnp.minimum(x0, gx0)
    cy = jnp.minimum(y0, gy0)
    dx = jnp.maximum(x1, gx1)
    dy = jnp.maximum(y1, gy1)
    area_c = (dx - cx) * (dy - cy)
    g = iou - (area_c - union) / area_c

    m = _reduce2(g, jnp.max)
    rows_f = lax.broadcasted_iota(jnp.int32, (H, W), 0).astype(jnp.float32)
    cols_f = lax.broadcasted_iota(jnp.int32, (H, W), 1).astype(jnp.float32)
    key = rows_f * 128.0 + cols_f
    cand = jnp.where(g == m, key, 3.4e38)
    kmin = _reduce2(cand, jnp.min)
    mi = jnp.floor(kmin * (1.0 / 128.0))
    mj = kmin - mi * 128.0

    rmask = (rows_f >= mi - float(R_LO)) & (rows_f <= jnp.minimum(mi + float(R_HI), float(H - 1)))
    cmask = (cols_f >= mj - float(R_LO)) & (cols_f <= jnp.minimum(mj + float(R_HI), float(W - 1)))
    mask = jnp.where(rmask & cmask, 1.0, 0.0)

    l1 = (jnp.abs(x0 - gx0) + jnp.abs(y0 - gy0)
          + jnp.abs(x1 - gx1) + jnp.abs(y1 - gy1)) * 0.25

    s_l1 = _reduce2(l1 * mask, jnp.sum)
    rn = jnp.minimum(mi + float(R_HI), float(H - 1)) - jnp.maximum(mi - float(R_LO), 0.0) + 1.0
    cn = jnp.minimum(mj + float(R_HI), float(W - 1)) - jnp.maximum(mj - float(R_LO), 0.0) + 1.0
    s_c = rn * cn
    s_g = s_c - _reduce2(g * mask, jnp.sum)

    valid = jnp.where(m > POS_IOU_THR, 1.0, 0.0)

    lane = lax.broadcasted_iota(jnp.int32, (1, 128), 1)
    return (jnp.where(lane == 0, 1.0, 0.0) * s_l1
            + jnp.where(lane == 1, 1.0, 0.0) * s_g
            + jnp.where(lane == 2, 1.0, 0.0) * s_c) * valid


def _loss_kernel(labels_ref, *refs):
    preds, gt_ref, out_ref = refs[:G], refs[G], refs[G + 1]
    s = pl.program_id(0)

    rows = []
    for i, p in enumerate(preds):
        base = (s * G + i) * 4
        rows.append(_one_gt(p, gt_ref[base + 0], gt_ref[base + 1],
                            gt_ref[base + 2], gt_ref[base + 3]))
    out_ref[...] = jnp.concatenate(rows, axis=0).reshape(1, G, 128)


def kernel(pred_boxes, gt_boxes, gt_labels):
    pred_t = jnp.moveaxis(pred_boxes, -1, 3).reshape(B, C, 4 * H, W)
    gt_flat = gt_boxes.reshape(-1)
    labels = gt_labels.astype(jnp.int32)

    labels_flat = labels.reshape(-1)

    pred_spec = [
        pl.BlockSpec((1, 1, 4 * H, W),
                     (lambda s, labels, i=i:
                      ((s * G + i) // N, labels[s * G + i], 0, 0)))
        for i in range(G)
    ]

    out = pl.pallas_call(
        _loss_kernel,
        grid_spec=pltpu.PrefetchScalarGridSpec(
            num_scalar_prefetch=1,
            grid=(B * N // G,),
            in_specs=pred_spec + [pl.BlockSpec(memory_space=pltpu.SMEM)],
            out_specs=pl.BlockSpec((1, G, 128),
                                   lambda s, labels: (s, 0, 0)),
        ),
        out_shape=jax.ShapeDtypeStruct((B * N // G, G, 128), jnp.float32),
        compiler_params=pltpu.CompilerParams(
            dimension_semantics=("arbitrary",),
        ),
        name="gtclass_loc_loss",
    )(labels_flat, *([pred_t] * G), gt_flat)

    l1_sum = jnp.sum(out[:, :, 0])
    g_sum = jnp.sum(out[:, :, 1])
    n_pos = jnp.sum(out[:, :, 2])
    denom = jnp.maximum(n_pos, 1.0)
    return LAMBDA_L1 * (l1_sum / denom) + LAMBDA_GIOU * (g_sum / denom)


# speedup vs baseline: 1.0296x; 1.0098x over previous
import jax
import jax.numpy as jnp
from jax import lax
from jax.experimental import pallas as pl
from jax.experimental.pallas import tpu as pltpu

B, C, H, W, N = 16, 80, 100, 100, 32
LAMBDA_L1, LAMBDA_GIOU, POS_IOU_THR = 1.0, 2.0, 0.3
R_LO, R_HI = 2, 1
G = 32


def _reduce2(x, op):
    r = op(x, axis=0, keepdims=True)
    return op(r, axis=1, keepdims=True)


def _one_gt(pred_ref, gscr, gx0, gy0, gx1, gy1):
    x0 = pred_ref[0, 0, 0::4, :]
    y0 = pred_ref[0, 0, 1::4, :]
    x1 = pred_ref[0, 0, 2::4, :]
    y1 = pred_ref[0, 0, 3::4, :]

    area_a = (x1 - x0) * (y1 - y0)
    area_b = (gx1 - gx0) * (gy1 - gy0)
    ltx = jnp.maximum(x0, gx0)
    lty = jnp.maximum(y0, gy0)
    rbx = jnp.minimum(x1, gx1)
    rby = jnp.minimum(y1, gy1)
    iw = jnp.maximum(rbx - ltx, 0.0)
    ih = jnp.maximum(rby - lty, 0.0)
    inter = iw * ih
    union = area_a + area_b - inter
    iou = inter / union
    cx = jnp.minimum(x0, gx0)
    cy = jnp.minimum(y0, gy0)
    dx = jnp.maximum(x1, gx1)
    dy = jnp.maximum(y1, gy1)
    area_c = (dx - cx) * (dy - cy)
    g = iou - (area_c - union) / area_c

    gscr[...] = g
    m = _reduce2(g, jnp.max)
    rows_f = lax.broadcasted_iota(jnp.int32, (H, W), 0).astype(jnp.float32)
    cols_f = lax.broadcasted_iota(jnp.int32, (H, W), 1).astype(jnp.float32)
    key = rows_f * 128.0 + cols_f
    cand = jnp.where(gscr[...] == m, key, 3.4e38)
    kmin = _reduce2(cand, jnp.min)
    mi = jnp.floor(kmin * (1.0 / 128.0))
    mj = kmin - mi * 128.0

    rmask = (rows_f >= mi - float(R_LO)) & (rows_f <= jnp.minimum(mi + float(R_HI), float(H - 1)))
    cmask = (cols_f >= mj - float(R_LO)) & (cols_f <= jnp.minimum(mj + float(R_HI), float(W - 1)))
    mask = jnp.where(rmask & cmask, 1.0, 0.0)

    l1 = (jnp.abs(x0 - gx0) + jnp.abs(y0 - gy0)
          + jnp.abs(x1 - gx1) + jnp.abs(y1 - gy1)) * 0.25

    s_l1 = _reduce2(l1 * mask, jnp.sum)
    rn = jnp.minimum(mi + float(R_HI), float(H - 1)) - jnp.maximum(mi - float(R_LO), 0.0) + 1.0
    cn = jnp.minimum(mj + float(R_HI), float(W - 1)) - jnp.maximum(mj - float(R_LO), 0.0) + 1.0
    s_c = rn * cn
    s_g = s_c - _reduce2(gscr[...] * mask, jnp.sum)

    valid = jnp.where(m > POS_IOU_THR, 1.0, 0.0)

    lane = lax.broadcasted_iota(jnp.int32, (1, 128), 1)
    return (jnp.where(lane == 0, 1.0, 0.0) * s_l1
            + jnp.where(lane == 1, 1.0, 0.0) * s_g
            + jnp.where(lane == 2, 1.0, 0.0) * s_c) * valid


def _loss_kernel(labels_ref, *refs):
    preds, gt_ref, out_ref = refs[:G], refs[G], refs[G + 1]
    s = pl.program_id(0)

    gscr = refs[G + 2]
    rows = []
    for i, p in enumerate(preds):
        base = (s * G + i) * 4
        rows.append(_one_gt(p, gscr.at[i], gt_ref[base + 0], gt_ref[base + 1],
                            gt_ref[base + 2], gt_ref[base + 3]))
    out_ref[...] = jnp.concatenate(rows, axis=0).reshape(1, G, 128)


def kernel(pred_boxes, gt_boxes, gt_labels):
    pred_t = jnp.moveaxis(pred_boxes, -1, 3).reshape(B, C, 4 * H, W)
    gt_flat = gt_boxes.reshape(-1)
    labels = gt_labels.astype(jnp.int32)

    labels_flat = labels.reshape(-1)

    pred_spec = [
        pl.BlockSpec((1, 1, 4 * H, W),
                     (lambda s, labels, i=i:
                      ((s * G + i) // N, labels[s * G + i], 0, 0)))
        for i in range(G)
    ]

    out = pl.pallas_call(
        _loss_kernel,
        grid_spec=pltpu.PrefetchScalarGridSpec(
            num_scalar_prefetch=1,
            grid=(B * N // G,),
            in_specs=pred_spec + [pl.BlockSpec(memory_space=pltpu.SMEM)],
            out_specs=pl.BlockSpec((1, G, 128),
                                   lambda s, labels: (s, 0, 0)),
            scratch_shapes=[pltpu.VMEM((G, H, W), jnp.float32)],
        ),
        out_shape=jax.ShapeDtypeStruct((B * N // G, G, 128), jnp.float32),
        compiler_params=pltpu.CompilerParams(
            dimension_semantics=("arbitrary",),
        ),
        name="gtclass_loc_loss",
    )(labels_flat, *([pred_t] * G), gt_flat)

    l1_sum = jnp.sum(out[:, :, 0])
    g_sum = jnp.sum(out[:, :, 1])
    n_pos = jnp.sum(out[:, :, 2])
    denom = jnp.maximum(n_pos, 1.0)
    return LAMBDA_L1 * (l1_sum / denom) + LAMBDA_GIOU * (g_sum / denom)
